# baseline (speedup 1.0000x reference)
import functools

import jax
import jax.numpy as jnp
from jax.experimental import pallas as pl
from jax.experimental.pallas import tpu as pltpu
from jax.experimental.pallas import tpu_sc as plsc

_TOP_K = 32
_NUM_KEYS = 100000
_PAD_W = 100352
_G = 128
_NUM_GROUPS = _PAD_W // _G
_GPAD = 896
_QB = 32
_NUM_Q = 1024
_NCAND = _TOP_K * _G
_QB3 = 256
_NROW = _NUM_Q * _TOP_K
_SC_CORES = 2
_SC_SUBCORES = 16
_SC_WORKERS = _SC_CORES * _SC_SUBCORES
_B_PER_W = _NROW // _SC_WORKERS
_CHUNK = 512
_NCHUNK = _B_PER_W // _CHUNK


def _knorm_body(k_ref, kn_ref):
    k = k_ref[...]
    n = jnp.sqrt(jnp.sum(k * k, axis=1, keepdims=True)) + 1e-8
    kn_ref[...] = k / n


def _k0_normalize(keys_pad):
    return pl.pallas_call(
        _knorm_body,
        grid=(8,),
        in_specs=[pl.BlockSpec((_PAD_W // 8, 16), lambda i: (i, 0))],
        out_specs=pl.BlockSpec((_PAD_W // 8, 16), lambda i: (i, 0)),
        out_shape=jax.ShapeDtypeStruct((_PAD_W, 16), jnp.float32),
    )(keys_pad)


def _groups_body(q_ref, ktn_ref, r_ref, s_ref):
    q = q_ref[...]
    qn = q / (jnp.sqrt(jnp.sum(q * q, axis=1, keepdims=True)) + 1e-8)
    s = jax.lax.dot_general(
        qn.astype(jnp.bfloat16),
        ktn_ref[...].astype(jnp.bfloat16),
        (((1,), (0,)), ((), ())),
        preferred_element_type=jnp.float32,
    )
    s_ref[...] = s.reshape(_QB, _NUM_GROUPS, _G)
    _FULL = _NUM_KEYS // _G
    l128 = jax.lax.broadcasted_iota(jnp.int32, (_QB, 1, _G), 2)
    s_ref[:, _FULL : _FULL + 1, :] = jnp.where(
        l128 >= _NUM_KEYS - _FULL * _G,
        -jnp.inf,
        s_ref[:, _FULL : _FULL + 1, :],
    )
    s_ref[:, _FULL + 1 :, :] = jnp.full(
        (_QB, _NUM_GROUPS - _FULL - 1, _G), -jnp.inf, jnp.float32
    )
    r = jnp.max(s_ref[...], axis=2)
    glane = jax.lax.broadcasted_iota(jnp.int32, (_QB, _GPAD), 1)
    r_ref[...] = jnp.where(
        glane < _NUM_GROUPS,
        jnp.pad(r, ((0, 0), (0, _GPAD - _NUM_GROUPS))),
        -jnp.inf,
    )


def _k1_groups(queries, ktn):
    return pl.pallas_call(
        _groups_body,
        grid=(_NUM_Q // _QB,),
        in_specs=[
            pl.BlockSpec((_QB, 16), lambda i: (i, 0)),
            pl.BlockSpec((16, _PAD_W), lambda i: (0, 0)),
        ],
        out_specs=[
            pl.BlockSpec((_QB, _GPAD), lambda i: (i, 0)),
            pl.BlockSpec((_QB, _NUM_GROUPS, _G), lambda i: (i, 0, 0)),
        ],
        out_shape=[
            jax.ShapeDtypeStruct((_NUM_Q, _GPAD), jnp.float32),
            jax.ShapeDtypeStruct((_NUM_Q, _NUM_GROUPS, _G), jnp.float32),
        ],
    )(queries, ktn)


def _phaseb_body(r_in_ref, gwin_ref, r_ref):
    r_ref[...] = r_in_ref[...]
    glane = jax.lax.broadcasted_iota(jnp.int32, (_NUM_Q, _GPAD), 1)
    col = jax.lax.broadcasted_iota(jnp.int32, (_NUM_Q, _TOP_K), 1)

    def body(i, gwin):
        r = r_ref[...]
        m = jnp.max(r, axis=1)
        g = jnp.min(jnp.where(r == m[:, None], glane, _GPAD), axis=1)
        r_ref[...] = jnp.where(glane == g[:, None], -jnp.inf, r)
        return jnp.where(col == i, g[:, None], gwin)

    gwin_ref[...] = jax.lax.fori_loop(
        0, _TOP_K, body, jnp.zeros((_NUM_Q, _TOP_K), jnp.int32)
    )


def _k1b_select_groups(r_full):
    return pl.pallas_call(
        _phaseb_body,
        out_shape=jax.ShapeDtypeStruct((_NUM_Q, _TOP_K), jnp.int32),
        scratch_shapes=[pltpu.VMEM((_NUM_Q, _GPAD), jnp.float32)],
    )(r_full)


def _gather_body(table_ref, idx_ref, out_ref, idx_v, rows_v, sem):
    wid = jax.lax.axis_index("s") * _SC_CORES + jax.lax.axis_index("c")
    w_base = wid * _B_PER_W

    @pl.loop(0, _NCHUNK)
    def _chunk(c):
        base = w_base + c * _CHUNK
        pltpu.sync_copy(idx_ref.at[pl.ds(base, _CHUNK)], idx_v)
        pltpu.async_copy(table_ref.at[idx_v], rows_v, sem).wait()
        pltpu.sync_copy(rows_v, out_ref.at[pl.ds(base, _CHUNK)])


def _k2_gather(sim_table, row_ids):
    mesh = plsc.VectorSubcoreMesh(core_axis_name="c", subcore_axis_name="s")
    k2 = functools.partial(
        pl.kernel,
        out_type=jax.ShapeDtypeStruct((_NROW, _G), jnp.float32),
        mesh=mesh,
        scratch_types=[
            pltpu.VMEM((_CHUNK,), jnp.int32),
            pltpu.VMEM((_CHUNK, _G), jnp.float32),
            pltpu.SemaphoreType.DMA,
        ],
    )(_gather_body)
    return k2(sim_table, row_ids)


def _final_body(sc_ref, cid_ref, vals_ref, idx_ref, s_ref):
    s_ref[...] = sc_ref[...]
    cid = cid_ref[...]
    col = jax.lax.broadcasted_iota(jnp.int32, (_QB3, _TOP_K), 1)
    big = jnp.int32(1 << 30)

    def body(i, carry):
        vals, idxs = carry
        s = s_ref[...]
        m = jnp.max(s, axis=1)
        wid_ = jnp.min(jnp.where(s == m[:, None], cid, big), axis=1)
        s_ref[...] = jnp.where(cid == wid_[:, None], -jnp.inf, s)
        vals = jnp.where(col == i, m[:, None], vals)
        idxs = jnp.where(col == i, wid_[:, None], idxs)
        return vals, idxs

    vals, idxs = jax.lax.fori_loop(
        0,
        _TOP_K,
        body,
        (
            jnp.zeros((_QB3, _TOP_K), jnp.float32),
            jnp.zeros((_QB3, _TOP_K), jnp.int32),
        ),
    )
    vals_ref[...] = vals
    idx_ref[...] = idxs


def _k3_final(s_cand, cand_ids2):
    return pl.pallas_call(
        _final_body,
        grid=(_NUM_Q // _QB3,),
        in_specs=[
            pl.BlockSpec((_QB3, _NCAND), lambda i: (i, 0)),
            pl.BlockSpec((_QB3, _NCAND), lambda i: (i, 0)),
        ],
        out_specs=[
            pl.BlockSpec((_QB3, _TOP_K), lambda i: (i, 0)),
            pl.BlockSpec((_QB3, _TOP_K), lambda i: (i, 0)),
        ],
        out_shape=[
            jax.ShapeDtypeStruct((_NUM_Q, _TOP_K), jnp.float32),
            jax.ShapeDtypeStruct((_NUM_Q, _TOP_K), jnp.int32),
        ],
        scratch_shapes=[pltpu.VMEM((_QB3, _NCAND), jnp.float32)],
    )(s_cand, cand_ids2)


@jax.jit
def kernel(queries, keys):
    keys_pad = jnp.pad(keys, ((0, _PAD_W - _NUM_KEYS), (0, 0)))
    kn = _k0_normalize(keys_pad)
    ktn = kn.T
    r_full, s3d = _k1_groups(queries, ktn)
    gwin = _k1b_select_groups(r_full)
    sim_table = s3d.reshape(_NUM_Q * _NUM_GROUPS, _G)
    qid = jnp.arange(_NUM_Q, dtype=jnp.int32)[:, None]
    row_ids = (qid * _NUM_GROUPS + gwin).reshape(_NROW)
    gs = _k2_gather(sim_table, row_ids)
    s_cand = gs.reshape(_NUM_Q, _NCAND)
    cand_ids2 = (
        gwin[:, :, None] * _G + jnp.arange(_G, dtype=jnp.int32)[None, None, :]
    ).reshape(_NUM_Q, _NCAND)
    vals, idxs = _k3_final(s_cand, cand_ids2)
    return vals, idxs
